# Initial kernel scaffold; baseline (speedup 1.0000x reference)
import jax
import jax.numpy as jnp
from jax.experimental import pallas as pl


def kernel(logits, label, mean_expand):
    raise NotImplementedError("write your pallas kernel here")


# speedup vs baseline: 1.1662x; 1.1662x over previous
import functools

import jax
import jax.numpy as jnp
from jax import lax
from jax.experimental import pallas as pl
from jax.experimental.pallas import tpu as pltpu
from jax.experimental.pallas import tpu_sc as plsc


def _sc_partial_sumsq(logits, label, table):
    B, P = logits.shape
    info = plsc.get_sparse_core_info()
    NC, NS, LN = info.num_cores, info.num_subcores, info.num_lanes
    NW = NC * NS
    rows_per = B // NW
    CH = 128
    nch = rows_per // CH

    mesh = plsc.VectorSubcoreMesh(core_axis_name="c", subcore_axis_name="s")

    @functools.partial(
        pl.kernel,
        mesh=mesh,
        out_type=jax.ShapeDtypeStruct((B, LN), jnp.float32),
        scratch_types=[
            pltpu.VMEM((CH,), jnp.int32),
            pltpu.VMEM((CH, P), jnp.float32),
            pltpu.VMEM((CH, P), jnp.float32),
            pltpu.VMEM((CH, LN), jnp.float32),
            pltpu.SemaphoreType.DMA,
        ],
    )
    def k(logits_hbm, label_hbm, table_hbm, out_hbm, idx_v, rows_v, log_v,
          out_v, sem):
        wid = lax.axis_index("s") * NC + lax.axis_index("c")
        base = wid * rows_per

        def chunk(ci, carry):
            cbase = base + ci * CH
            pltpu.sync_copy(label_hbm.at[pl.ds(cbase, CH)], idx_v)
            gather = pltpu.async_copy(table_hbm.at[idx_v], rows_v, sem)
            pltpu.sync_copy(logits_hbm.at[pl.ds(cbase, CH)], log_v)
            gather.wait()

            def row(r, carry2):
                acc = jnp.zeros((LN,), jnp.float32)
                for j in range(P // LN):
                    d = (log_v[r, pl.ds(j * LN, LN)]
                         - rows_v[r, pl.ds(j * LN, LN)])
                    acc = acc + d * d
                out_v[r, :] = acc
                return carry2

            lax.fori_loop(0, CH, row, 0, unroll=2)
            pltpu.sync_copy(out_v, out_hbm.at[pl.ds(cbase, CH)])
            return carry

        lax.fori_loop(0, nch, chunk, 0)

    return k(logits, label, table)


def _tc_finish(partial):
    B, LN = partial.shape
    x = partial.reshape(B * LN // 128, 128)

    def body(x_ref, o_ref):
        xv = x_ref[...]
        grp = lax.broadcasted_iota(jnp.int32, (128, 128 // LN), 0) // LN
        col = lax.broadcasted_iota(jnp.int32, (128, 128 // LN), 1)
        fold = (grp == col).astype(jnp.float32)
        sumsq = jnp.dot(xv, fold, preferred_element_type=jnp.float32)
        o_ref[0, 0] = jnp.sum(jnp.sqrt(sumsq)) / B

    out = pl.pallas_call(
        body,
        out_shape=jax.ShapeDtypeStruct((1, 1), jnp.float32),
        out_specs=pl.BlockSpec(memory_space=pltpu.SMEM),
    )(x)
    return out[0, 0]


def kernel(logits, label, mean_expand):
    label = label.astype(jnp.int32)
    partial = _sc_partial_sumsq(logits, label, mean_expand)
    return _tc_finish(partial)
